# baseline (speedup 1.0000x reference)
import jax
import jax.numpy as jnp
from jax import lax
from jax.experimental import pallas as pl
from jax.experimental.pallas import tpu as pltpu
from jax.experimental.pallas import tpu_sc as plsc

N = 6890
F = 13776
NP = 6912
FP = 13824
NT = 16
VSL = NP // NT
FSL = FP // NT
HCP = 1556
HA = HCP // 2
BIG = 1e10
ROWS = 128


def _nn_body(v_ref, vt_ref, geo_ref, iv_ref, sel_ref, bp_ref, ip_ref,
             lh_ref, rh_ref, vmin_ref, amin_ref, gmin_ref,
             vx_ref, vy_ref, vz_ref, scal_ref):
    i = pl.program_id(0)
    vb = v_ref[...]
    vt = vt_ref[...]
    geo = geo_ref[...]
    sqi = (vb[:, 0:1] * vb[:, 0:1] + vb[:, 1:2] * vb[:, 1:2]
           + vb[:, 2:3] * vb[:, 2:3])
    sqj = (vt[0:1, :] * vt[0:1, :] + vt[1:2, :] * vt[1:2, :]
           + vt[2:3, :] * vt[2:3, :])
    dot = lax.dot_general(vb, vt, (((1,), (0,)), ((), ())),
                          preferred_element_type=jnp.float32)
    d2 = jnp.maximum((sqi + sqj) - 2.0 * dot, 0.0)
    masked = jnp.where(geo < 0.3, BIG, d2)
    m = jnp.min(masked, axis=1, keepdims=True)
    col = lax.broadcasted_iota(jnp.int32, masked.shape, 1)
    idx = jnp.min(jnp.where(masked == m, col, jnp.int32(2**30)),
                  axis=1, keepdims=True)
    row = i * ROWS + lax.broadcasted_iota(jnp.int32, (ROWS, 1), 0)
    ok = row < N
    vmin_ref[...] = jnp.where(
        ok, jnp.where(m >= 1e9, BIG, jnp.sqrt(m + 1e-12)), 0.0)
    amin_ref[...] = jnp.where(ok, idx, 0)
    gm = jnp.min(jnp.where(sel_ref[...] > 0.0, geo, BIG),
                 axis=1, keepdims=True)
    gmin_ref[...] = jnp.where(ok, gm, 0.0)
    vx_ref[...] = vb[:, 0:1]
    vy_ref[...] = vb[:, 1:2]
    vz_ref[...] = vb[:, 2:3]
    dv = iv_ref[...] - vb
    ov = jnp.sqrt(dv[:, 0:1] * dv[:, 0:1] + dv[:, 1:2] * dv[:, 1:2]
                  + dv[:, 2:3] * dv[:, 2:3])
    ow = (2.0 * gm) ** 2
    part = jnp.sum(jnp.where(ok, ov * ow, 0.0))
    lanev = lax.broadcasted_iota(jnp.int32, (1, 128), 1)

    @pl.when(i == 0)
    def _():
        bp = bp_ref[...]
        ip = ip_ref[...]
        pose = jnp.sum((bp - ip) ** 2)
        hand = 0.001 * (jnp.sum(lh_ref[...] ** 2) + jnp.sum(rh_ref[...] ** 2))
        scal_ref[...] = jnp.where(lanev == 0, pose + hand, 0.0)

    scal_ref[...] += jnp.where(lanev == 0, 2.0 * part, 0.0)


def _nn_pass(v, vt, geo, iv, sel, bp, ip, lh, rh):
    grid = NP // ROWS
    return pl.pallas_call(
        _nn_body,
        grid=(grid,),
        in_specs=[
            pl.BlockSpec((ROWS, 3), lambda i: (i, 0)),
            pl.BlockSpec((3, N), lambda i: (0, 0)),
            pl.BlockSpec((ROWS, N), lambda i: (i, 0)),
            pl.BlockSpec((ROWS, 3), lambda i: (i, 0)),
            pl.BlockSpec((1, N), lambda i: (0, 0)),
            pl.BlockSpec((1, 63), lambda i: (0, 0)),
            pl.BlockSpec((1, 63), lambda i: (0, 0)),
            pl.BlockSpec((1, 45), lambda i: (0, 0)),
            pl.BlockSpec((1, 45), lambda i: (0, 0)),
        ],
        out_specs=[
            pl.BlockSpec((ROWS, 1), lambda i: (i, 0)),
            pl.BlockSpec((ROWS, 1), lambda i: (i, 0)),
            pl.BlockSpec((ROWS, 1), lambda i: (i, 0)),
            pl.BlockSpec((ROWS, 1), lambda i: (i, 0)),
            pl.BlockSpec((ROWS, 1), lambda i: (i, 0)),
            pl.BlockSpec((ROWS, 1), lambda i: (i, 0)),
            pl.BlockSpec((1, 128), lambda i: (0, 0)),
        ],
        out_shape=[
            jax.ShapeDtypeStruct((NP, 1), jnp.float32),
            jax.ShapeDtypeStruct((NP, 1), jnp.int32),
            jax.ShapeDtypeStruct((NP, 1), jnp.float32),
            jax.ShapeDtypeStruct((NP, 1), jnp.float32),
            jax.ShapeDtypeStruct((NP, 1), jnp.float32),
            jax.ShapeDtypeStruct((NP, 1), jnp.float32),
            jax.ShapeDtypeStruct((1, 128), jnp.float32),
        ],
    )(v, vt, geo, iv, sel, bp, ip, lh, rh)


def _tanh(x):
    e = jnp.exp(2.0 * x)
    return 1.0 - 2.0 / (e + 1.0)


def _rsqrt(s):
    sc = jnp.maximum(s, 1e-30)
    i = plsc.bitcast(sc, jnp.int32)
    y = plsc.bitcast(jnp.int32(0x5F3759DF) - lax.shift_right_logical(i, 1),
                     jnp.float32)
    for _ in range(4):
        y = y * (1.5 - 0.5 * sc * y * y)
    return jnp.where(s < 1e-30, 0.0, y)


def _lane_pack(scalars):
    lane = lax.iota(jnp.int32, 16)
    out = jnp.zeros((16,), jnp.float32)
    for k, s in enumerate(scalars):
        out = out + jnp.where(lane == k, s, 0.0)
    return out


def _sc_tail_body(vx_h, vy_h, vz_h, ff_h, v2v_h, amin_h, gmin_h,
                  ds_h, hp_h, hw_h, base_h, zf_h, zi_h,
                  loss_o, inside_o,
                  vx_v, vy_v, vz_v, ax_v, ay_v, az_v,
                  v2v_v, amin_v, gmin_v, inside_v,
                  f_v, ds_v, hp_v, hw_v,
                  rx_v, ry_v, rz_v, tmp_v,
                  buf_i, buf_f, pbuf_v, base_v,
                  sax, say, saz, svx, svy, svz, sins, sparts):
    cid = lax.axis_index("c")
    w = lax.axis_index("s")

    @pl.when(cid == 0)
    def _core0():
        pltpu.sync_copy(vx_h, vx_v)
        pltpu.sync_copy(vy_h, vy_v)
        pltpu.sync_copy(vz_h, vz_v)
        pltpu.sync_copy(ff_h.at[pl.ds(w * (3 * FSL), 3 * FSL)], f_v)
        pltpu.sync_copy(v2v_h, v2v_v)
        pltpu.sync_copy(amin_h, amin_v)
        pltpu.sync_copy(gmin_h, gmin_v)
        pltpu.sync_copy(ds_h.at[pl.ds(w * 64, 64)], ds_v)
        pltpu.sync_copy(hp_h, hp_v)
        pltpu.sync_copy(hw_h, hw_v)
        pltpu.sync_copy(zf_h, ax_v)
        pltpu.sync_copy(zf_h, ay_v)
        pltpu.sync_copy(zf_h, az_v)

        @pl.when(w == 0)
        def _():
            pltpu.sync_copy(zi_h, sins)
            pltpu.sync_copy(base_h.at[pl.ds(0, 16)], base_v)

        i3 = lax.iota(jnp.int32, 16) * 3

        def face_step(k, carry):
            b3 = k * 48
            f0 = plsc.load_gather(f_v, [b3 + i3])
            f1 = plsc.load_gather(f_v, [b3 + i3 + 1])
            f2 = plsc.load_gather(f_v, [b3 + i3 + 2])
            x0 = plsc.load_gather(vx_v, [f0])
            y0 = plsc.load_gather(vy_v, [f0])
            z0 = plsc.load_gather(vz_v, [f0])
            x1 = plsc.load_gather(vx_v, [f1])
            y1 = plsc.load_gather(vy_v, [f1])
            z1 = plsc.load_gather(vz_v, [f1])
            x2 = plsc.load_gather(vx_v, [f2])
            y2 = plsc.load_gather(vy_v, [f2])
            z2 = plsc.load_gather(vz_v, [f2])
            e1x, e1y, e1z = x1 - x0, y1 - y0, z1 - z0
            e2x, e2y, e2z = x2 - x0, y2 - y0, z2 - z0
            cx = e1y * e2z - e1z * e2y
            cy = e1z * e2x - e1x * e2z
            cz = e1x * e2y - e1y * e2x
            plsc.addupdate_scatter(ax_v, [f0], cx)
            plsc.addupdate_scatter(ax_v, [f1], cx)
            plsc.addupdate_scatter(ax_v, [f2], cx)
            plsc.addupdate_scatter(ay_v, [f0], cy)
            plsc.addupdate_scatter(ay_v, [f1], cy)
            plsc.addupdate_scatter(ay_v, [f2], cy)
            plsc.addupdate_scatter(az_v, [f0], cz)
            plsc.addupdate_scatter(az_v, [f1], cz)
            plsc.addupdate_scatter(az_v, [f2], cz)
            return carry

        lax.fori_loop(0, FSL // 16, face_step, 0)
        pltpu.sync_copy(ax_v, sax.at[pl.ds(w * NP, NP)])
        pltpu.sync_copy(ay_v, say.at[pl.ds(w * NP, NP)])
        pltpu.sync_copy(az_v, saz.at[pl.ds(w * NP, NP)])
        plsc.subcore_barrier()

        off = w * VSL
        pltpu.sync_copy(sax.at[pl.ds(off, VSL)], rx_v)
        pltpu.sync_copy(say.at[pl.ds(off, VSL)], ry_v)
        pltpu.sync_copy(saz.at[pl.ds(off, VSL)], rz_v)
        for r in range(1, NT):
            for src, dst in ((sax, rx_v), (say, ry_v), (saz, rz_v)):
                pltpu.sync_copy(src.at[pl.ds(r * NP + off, VSL)], tmp_v)

                def add_step(k, carry, dst=dst):
                    b = k * 16
                    dst[pl.ds(b, 16)] = dst[pl.ds(b, 16)] + tmp_v[pl.ds(b, 16)]
                    return carry

                lax.fori_loop(0, VSL // 16, add_step, 0)

        def norm_step(k, carry):
            b = k * 16
            x = rx_v[pl.ds(b, 16)]
            y = ry_v[pl.ds(b, 16)]
            z = rz_v[pl.ds(b, 16)]
            s = x * x + y * y + z * z
            r = _rsqrt(s)
            inv = 1.0 / (s * r + 1e-12)
            rx_v[pl.ds(b, 16)] = x * inv
            ry_v[pl.ds(b, 16)] = y * inv
            rz_v[pl.ds(b, 16)] = z * inv
            return carry

        lax.fori_loop(0, VSL // 16, norm_step, 0)
        pltpu.sync_copy(rx_v, svx.at[pl.ds(off, VSL)])
        pltpu.sync_copy(ry_v, svy.at[pl.ds(off, VSL)])
        pltpu.sync_copy(rz_v, svz.at[pl.ds(off, VSL)])
        plsc.subcore_barrier()
        pltpu.sync_copy(svx, ax_v)
        pltpu.sync_copy(svy, ay_v)
        pltpu.sync_copy(svz, az_v)

        def ds_step(k, carry):
            cnum, cden = carry
            b = k * 16
            dsi = ds_v[pl.ds(b, 16)]
            j = plsc.load_gather(amin_v, [dsi])
            nx = plsc.load_gather(ax_v, [j])
            ny = plsc.load_gather(ay_v, [j])
            nz = plsc.load_gather(az_v, [j])
            dx = plsc.load_gather(vx_v, [dsi]) - plsc.load_gather(vx_v, [j])
            dy = plsc.load_gather(vy_v, [dsi]) - plsc.load_gather(vy_v, [j])
            dz = plsc.load_gather(vz_v, [dsi]) - plsc.load_gather(vz_v, [j])
            ext = nx * dx + ny * dy + nz * dz >= 0.0
            ins = jnp.where(ext, 0, 1).astype(jnp.int32)
            buf_i[...] = ins
            pltpu.sync_copy(buf_i, sins.at[dsi])
            insf = ins.astype(jnp.float32)
            v2vd = plsc.load_gather(v2v_v, [dsi])
            gmd = plsc.load_gather(gmin_v, [dsi])
            val = 0.04 * (1.0 / (5.0 * gmd + 1.0)) * _tanh(v2vd / 0.04)
            cnum = cnum + val * (1.0 - insf)
            cden = cden + (1.0 - insf)
            return cnum, cden

        z16 = jnp.zeros((16,), jnp.float32)
        cnum, cden = lax.fori_loop(0, 4, ds_step, (z16, z16))
        plsc.subcore_barrier()
        pltpu.sync_copy(sins, inside_v)

        def slice_step(k, carry):
            inum, iden, anum, aden = carry
            b = w * VSL + k * 16
            pos = b + lax.iota(jnp.int32, 16)
            valid = pos < N
            insf = inside_v[pl.ds(b, 16)].astype(jnp.float32)
            v2vc = v2v_v[pl.ds(b, 16)]
            inum = inum + insf * _tanh(v2vc / 0.06)
            iden = iden + insf
            j = amin_v[pl.ds(b, 16)]
            dotn = (ax_v[pl.ds(b, 16)] * plsc.load_gather(ax_v, [j])
                    + ay_v[pl.ds(b, 16)] * plsc.load_gather(ay_v, [j])
                    + az_v[pl.ds(b, 16)] * plsc.load_gather(az_v, [j]))
            amask = (v2vc < 0.01) & valid
            anum = anum + jnp.where(amask, 1.0 + dotn, 0.0)
            aden = aden + jnp.where(amask, 1.0, 0.0)
            return inum, iden, anum, aden

        inum, iden, anum, aden = lax.fori_loop(
            0, VSL // 16, slice_step, (z16, z16, z16, z16))

        def hand_step(k, carry):
            lin, lid, lon, lod, rin, rid, ron, rod = carry
            pos = w * 64 + k * 16 + lax.iota(jnp.int32, 16)
            ok = jnp.where(pos < HA, 1.0, 0.0)
            posc = jnp.minimum(pos, HA - 1)
            li = plsc.load_gather(hp_v, [posc])
            ri = plsc.load_gather(hp_v, [posc + HA])
            lhv = plsc.load_gather(inside_v, [li]).astype(jnp.float32)
            rhv = plsc.load_gather(inside_v, [ri]).astype(jnp.float32)
            lv = plsc.load_gather(v2v_v, [li])
            rv = plsc.load_gather(v2v_v, [ri])
            lwc = 0.1 * (1.0 - plsc.load_gather(hw_v, [posc])) + 0.9
            rwc = 0.1 * (1.0 - plsc.load_gather(hw_v, [posc + HA])) + 0.9
            lin = lin + ok * lhv * _tanh(lv / 0.02)
            lid = lid + ok * lhv
            lon = lon + ok * (1.0 - lhv) * lwc * _tanh(lv / 0.01)
            lod = lod + ok * (1.0 - lhv)
            rin = rin + ok * rhv * _tanh(rv / 0.02)
            rid = rid + ok * rhv
            ron = ron + ok * (1.0 - rhv) * rwc * _tanh(rv / 0.01)
            rod = rod + ok * (1.0 - rhv)
            return lin, lid, lon, lod, rin, rid, ron, rod

        hand = lax.fori_loop(0, 4, hand_step, (z16,) * 8)

        lin, lid, lon, lod, rin, rid, ron, rod = hand
        nums = _lane_pack([jnp.sum(x) for x in
                           (inum, anum, cnum, lin, rin, lon, ron)])
        dens = _lane_pack([jnp.sum(x) for x in
                           (iden, aden, cden, lid, rid, lod, rod)])
        buf_f[...] = nums
        pltpu.sync_copy(buf_f, sparts.at[pl.ds(w * 32, 16)])
        buf_f[...] = dens
        pltpu.sync_copy(buf_f, sparts.at[pl.ds(w * 32 + 16, 16)])
        plsc.subcore_barrier()

        @pl.when(w == 0)
        def _final():
            pltpu.sync_copy(sparts, pbuf_v)
            numv = jnp.zeros((16,), jnp.float32)
            denv = jnp.zeros((16,), jnp.float32)
            for r in range(NT):
                numv = numv + pbuf_v[pl.ds(r * 32, 16)]
                denv = denv + pbuf_v[pl.ds(r * 32 + 16, 16)]
            quot = numv / jnp.maximum(denv, 1.0)
            coef = _lane_pack([0.5 * 0.07, 0.001, 5.0,
                               0.2 * 0.5 * 0.023, 0.2 * 0.5 * 0.023,
                               0.2 * 0.5 * 0.01, 0.2 * 0.5 * 0.01])
            loss = jnp.sum(coef * quot) + jnp.sum(base_v[...])
            lane = lax.iota(jnp.int32, 16)
            buf_f[...] = jnp.where(lane == 0, loss, 0.0)
            pltpu.sync_copy(buf_f, loss_o)
            pltpu.sync_copy(sins, inside_o)


def _sc_tail(vx, vy, vz, ff, v2v, amin, gmin, ds, hp, hw, base128, zf, zi):
    mesh = plsc.VectorSubcoreMesh(core_axis_name="c", subcore_axis_name="s")
    fn = pl.kernel(
        _sc_tail_body, mesh=mesh,
        compiler_params=pltpu.CompilerParams(needs_layout_passes=False),
        out_type=[jax.ShapeDtypeStruct((16,), jnp.float32),
                  jax.ShapeDtypeStruct((NP,), jnp.int32)],
        scratch_types=[
            pltpu.VMEM((NP,), jnp.float32),
            pltpu.VMEM((NP,), jnp.float32),
            pltpu.VMEM((NP,), jnp.float32),
            pltpu.VMEM((NP,), jnp.float32),
            pltpu.VMEM((NP,), jnp.float32),
            pltpu.VMEM((NP,), jnp.float32),
            pltpu.VMEM((NP,), jnp.float32),
            pltpu.VMEM((NP,), jnp.int32),
            pltpu.VMEM((NP,), jnp.float32),
            pltpu.VMEM((NP,), jnp.int32),
            pltpu.VMEM((3 * FSL,), jnp.int32),
            pltpu.VMEM((64,), jnp.int32),
            pltpu.VMEM((1600,), jnp.int32),
            pltpu.VMEM((1600,), jnp.float32),
            pltpu.VMEM((VSL,), jnp.float32),
            pltpu.VMEM((VSL,), jnp.float32),
            pltpu.VMEM((VSL,), jnp.float32),
            pltpu.VMEM((VSL,), jnp.float32),
            pltpu.VMEM((16,), jnp.int32),
            pltpu.VMEM((16,), jnp.float32),
            pltpu.VMEM((NT * 32,), jnp.float32),
            pltpu.VMEM((16,), jnp.float32),
            pltpu.VMEM_SHARED((NT * NP,), jnp.float32),
            pltpu.VMEM_SHARED((NT * NP,), jnp.float32),
            pltpu.VMEM_SHARED((NT * NP,), jnp.float32),
            pltpu.VMEM_SHARED((NP,), jnp.float32),
            pltpu.VMEM_SHARED((NP,), jnp.float32),
            pltpu.VMEM_SHARED((NP,), jnp.float32),
            pltpu.VMEM_SHARED((NP,), jnp.int32),
            pltpu.VMEM_SHARED((NT * 32,), jnp.float32),
        ],
    )
    return fn(vx, vy, vz, ff, v2v, amin, gmin, ds, hp, hw, base128, zf, zi)


def kernel(vertices, init_verts, body_pose, init_pose, left_hand_pose,
           right_hand_pose, geodist, hand_contact_prior_weights,
           ds, hand_contact_prior, faces, init_verts_in_contact):
    v = vertices[0]
    iv = init_verts[0]
    sel = jnp.zeros((1, N), jnp.float32).at[0, init_verts_in_contact].set(1.0)
    vmin, amin, gmin, vx, vy, vz, scal = _nn_pass(
        v, v.T, geodist, iv, sel, body_pose, init_pose,
        left_hand_pose, right_hand_pose)

    ff = jnp.zeros((3 * FP,), jnp.int32).at[:3 * F].set(faces.reshape(-1))
    hp = jnp.zeros((1600,), jnp.int32).at[:HCP].set(hand_contact_prior)
    hw = jnp.zeros((1600,), jnp.float32).at[:HCP].set(
        hand_contact_prior_weights)
    zf = jnp.zeros((NP,), jnp.float32)
    zi = jnp.zeros((NP,), jnp.int32)

    loss16, inside_np = _sc_tail(
        vx.reshape(NP), vy.reshape(NP), vz.reshape(NP), ff,
        vmin.reshape(NP), amin.reshape(NP), gmin.reshape(NP),
        ds.astype(jnp.int32), hp, hw, scal.reshape(128), zf, zi)
    return (loss16[0], inside_np[:N].astype(bool))


# speedup vs baseline: 1.0740x; 1.0740x over previous
import jax
import jax.numpy as jnp
from jax import lax
from jax.experimental import pallas as pl
from jax.experimental.pallas import tpu as pltpu
from jax.experimental.pallas import tpu_sc as plsc

N = 6890
F = 13776
NP = 6912
FP = 13824
NT = 16
VSL = NP // NT
FSL = FP // NT
HCP = 1556
HA = HCP // 2
BIG = 1e10
ROWS = 256


def _nn_body(v_ref, vt_ref, geo_ref, iv_ref, sel_ref, bp_ref, ip_ref,
             lh_ref, rh_ref, vmin_ref, amin_ref, gmin_ref,
             vx_ref, vy_ref, vz_ref, scal_ref):
    i = pl.program_id(0)
    vb = v_ref[...]
    vt = vt_ref[...]
    geo = geo_ref[...]
    sqi = (vb[:, 0:1] * vb[:, 0:1] + vb[:, 1:2] * vb[:, 1:2]
           + vb[:, 2:3] * vb[:, 2:3])
    sqj = (vt[0:1, :] * vt[0:1, :] + vt[1:2, :] * vt[1:2, :]
           + vt[2:3, :] * vt[2:3, :])
    dot = lax.dot_general(vb, vt, (((1,), (0,)), ((), ())),
                          preferred_element_type=jnp.float32)
    d2 = jnp.maximum((sqi + sqj) - 2.0 * dot, 0.0)
    masked = jnp.where(geo < 0.3, BIG, d2)
    m = jnp.min(masked, axis=1, keepdims=True)
    col = lax.broadcasted_iota(jnp.int32, masked.shape, 1)
    idx = jnp.min(jnp.where(masked == m, col, jnp.int32(2**30)),
                  axis=1, keepdims=True)
    row = i * ROWS + lax.broadcasted_iota(jnp.int32, (ROWS, 1), 0)
    ok = row < N
    vmin_ref[...] = jnp.where(
        ok, jnp.where(m >= 1e9, BIG, jnp.sqrt(m + 1e-12)), 0.0)
    amin_ref[...] = jnp.where(ok, idx, 0)
    gm = jnp.min(jnp.where(sel_ref[...] > 0.0, geo, BIG),
                 axis=1, keepdims=True)
    gmin_ref[...] = jnp.where(ok, gm, 0.0)
    vx_ref[...] = vb[:, 0:1]
    vy_ref[...] = vb[:, 1:2]
    vz_ref[...] = vb[:, 2:3]
    dv = iv_ref[...] - vb
    ov = jnp.sqrt(dv[:, 0:1] * dv[:, 0:1] + dv[:, 1:2] * dv[:, 1:2]
                  + dv[:, 2:3] * dv[:, 2:3])
    ow = (2.0 * gm) ** 2
    part = jnp.sum(jnp.where(ok, ov * ow, 0.0))
    lanev = lax.broadcasted_iota(jnp.int32, (1, 128), 1)

    @pl.when(i == 0)
    def _():
        bp = bp_ref[...]
        ip = ip_ref[...]
        pose = jnp.sum((bp - ip) ** 2)
        hand = 0.001 * (jnp.sum(lh_ref[...] ** 2) + jnp.sum(rh_ref[...] ** 2))
        scal_ref[...] = jnp.where(lanev == 0, pose + hand, 0.0)

    scal_ref[...] += jnp.where(lanev == 0, 2.0 * part, 0.0)


def _nn_pass(v, vt, geo, iv, sel, bp, ip, lh, rh):
    grid = NP // ROWS
    return pl.pallas_call(
        _nn_body,
        grid=(grid,),
        in_specs=[
            pl.BlockSpec((ROWS, 3), lambda i: (i, 0)),
            pl.BlockSpec((3, N), lambda i: (0, 0)),
            pl.BlockSpec((ROWS, N), lambda i: (i, 0)),
            pl.BlockSpec((ROWS, 3), lambda i: (i, 0)),
            pl.BlockSpec((1, N), lambda i: (0, 0)),
            pl.BlockSpec((1, 63), lambda i: (0, 0)),
            pl.BlockSpec((1, 63), lambda i: (0, 0)),
            pl.BlockSpec((1, 45), lambda i: (0, 0)),
            pl.BlockSpec((1, 45), lambda i: (0, 0)),
        ],
        out_specs=[
            pl.BlockSpec((ROWS, 1), lambda i: (i, 0)),
            pl.BlockSpec((ROWS, 1), lambda i: (i, 0)),
            pl.BlockSpec((ROWS, 1), lambda i: (i, 0)),
            pl.BlockSpec((ROWS, 1), lambda i: (i, 0)),
            pl.BlockSpec((ROWS, 1), lambda i: (i, 0)),
            pl.BlockSpec((ROWS, 1), lambda i: (i, 0)),
            pl.BlockSpec((1, 128), lambda i: (0, 0)),
        ],
        out_shape=[
            jax.ShapeDtypeStruct((NP, 1), jnp.float32),
            jax.ShapeDtypeStruct((NP, 1), jnp.int32),
            jax.ShapeDtypeStruct((NP, 1), jnp.float32),
            jax.ShapeDtypeStruct((NP, 1), jnp.float32),
            jax.ShapeDtypeStruct((NP, 1), jnp.float32),
            jax.ShapeDtypeStruct((NP, 1), jnp.float32),
            jax.ShapeDtypeStruct((1, 128), jnp.float32),
        ],
    )(v, vt, geo, iv, sel, bp, ip, lh, rh)


def _tanh(x):
    e = jnp.exp(2.0 * x)
    return 1.0 - 2.0 / (e + 1.0)


def _rsqrt(s):
    sc = jnp.maximum(s, 1e-30)
    i = plsc.bitcast(sc, jnp.int32)
    y = plsc.bitcast(jnp.int32(0x5F3759DF) - lax.shift_right_logical(i, 1),
                     jnp.float32)
    for _ in range(4):
        y = y * (1.5 - 0.5 * sc * y * y)
    return jnp.where(s < 1e-30, 0.0, y)


def _lane_pack(scalars):
    lane = lax.iota(jnp.int32, 16)
    out = jnp.zeros((16,), jnp.float32)
    for k, s in enumerate(scalars):
        out = out + jnp.where(lane == k, s, 0.0)
    return out


def _sc_tail_body(vx_h, vy_h, vz_h, ff_h, v2v_h, amin_h, gmin_h,
                  ds_h, hp_h, hw_h, base_h, zf_h, zi_h,
                  loss_o, inside_o,
                  vx_v, vy_v, vz_v, ax_v, ay_v, az_v,
                  v2v_v, amin_v, gmin_v, inside_v,
                  f_v, ds_v, hp_v, hw_v,
                  rx_v, ry_v, rz_v, tmp_v,
                  buf_i, buf_f, pbuf_v, base_v,
                  sax, say, saz, svx, svy, svz, sins, sparts):
    cid = lax.axis_index("c")
    w = lax.axis_index("s")

    @pl.when(cid == 0)
    def _core0():
        pltpu.sync_copy(vx_h, vx_v)
        pltpu.sync_copy(vy_h, vy_v)
        pltpu.sync_copy(vz_h, vz_v)
        pltpu.sync_copy(ff_h.at[pl.ds(w * (3 * FSL), 3 * FSL)], f_v)
        pltpu.sync_copy(v2v_h, v2v_v)
        pltpu.sync_copy(amin_h, amin_v)
        pltpu.sync_copy(gmin_h, gmin_v)
        pltpu.sync_copy(ds_h.at[pl.ds(w * 64, 64)], ds_v)
        pltpu.sync_copy(hp_h, hp_v)
        pltpu.sync_copy(hw_h, hw_v)
        pltpu.sync_copy(zf_h, ax_v)
        pltpu.sync_copy(zf_h, ay_v)
        pltpu.sync_copy(zf_h, az_v)

        @pl.when(w == 0)
        def _():
            pltpu.sync_copy(zi_h, sins)
            pltpu.sync_copy(base_h.at[pl.ds(0, 16)], base_v)

        i3 = lax.iota(jnp.int32, 16) * 3

        def face_step(k, carry):
            b3 = k * 48
            f0 = plsc.load_gather(f_v, [b3 + i3])
            f1 = plsc.load_gather(f_v, [b3 + i3 + 1])
            f2 = plsc.load_gather(f_v, [b3 + i3 + 2])
            x0 = plsc.load_gather(vx_v, [f0])
            y0 = plsc.load_gather(vy_v, [f0])
            z0 = plsc.load_gather(vz_v, [f0])
            x1 = plsc.load_gather(vx_v, [f1])
            y1 = plsc.load_gather(vy_v, [f1])
            z1 = plsc.load_gather(vz_v, [f1])
            x2 = plsc.load_gather(vx_v, [f2])
            y2 = plsc.load_gather(vy_v, [f2])
            z2 = plsc.load_gather(vz_v, [f2])
            e1x, e1y, e1z = x1 - x0, y1 - y0, z1 - z0
            e2x, e2y, e2z = x2 - x0, y2 - y0, z2 - z0
            cx = e1y * e2z - e1z * e2y
            cy = e1z * e2x - e1x * e2z
            cz = e1x * e2y - e1y * e2x
            plsc.addupdate_scatter(ax_v, [f0], cx)
            plsc.addupdate_scatter(ax_v, [f1], cx)
            plsc.addupdate_scatter(ax_v, [f2], cx)
            plsc.addupdate_scatter(ay_v, [f0], cy)
            plsc.addupdate_scatter(ay_v, [f1], cy)
            plsc.addupdate_scatter(ay_v, [f2], cy)
            plsc.addupdate_scatter(az_v, [f0], cz)
            plsc.addupdate_scatter(az_v, [f1], cz)
            plsc.addupdate_scatter(az_v, [f2], cz)
            return carry

        lax.fori_loop(0, FSL // 16, face_step, 0)
        pltpu.sync_copy(ax_v, sax.at[pl.ds(w * NP, NP)])
        pltpu.sync_copy(ay_v, say.at[pl.ds(w * NP, NP)])
        pltpu.sync_copy(az_v, saz.at[pl.ds(w * NP, NP)])
        plsc.subcore_barrier()

        off = w * VSL
        pltpu.sync_copy(sax.at[pl.ds(off, VSL)], rx_v)
        pltpu.sync_copy(say.at[pl.ds(off, VSL)], ry_v)
        pltpu.sync_copy(saz.at[pl.ds(off, VSL)], rz_v)
        for r in range(1, NT):
            for src, dst in ((sax, rx_v), (say, ry_v), (saz, rz_v)):
                pltpu.sync_copy(src.at[pl.ds(r * NP + off, VSL)], tmp_v)

                def add_step(k, carry, dst=dst):
                    b = k * 16
                    dst[pl.ds(b, 16)] = dst[pl.ds(b, 16)] + tmp_v[pl.ds(b, 16)]
                    return carry

                lax.fori_loop(0, VSL // 16, add_step, 0)

        def norm_step(k, carry):
            b = k * 16
            x = rx_v[pl.ds(b, 16)]
            y = ry_v[pl.ds(b, 16)]
            z = rz_v[pl.ds(b, 16)]
            s = x * x + y * y + z * z
            r = _rsqrt(s)
            inv = 1.0 / (s * r + 1e-12)
            rx_v[pl.ds(b, 16)] = x * inv
            ry_v[pl.ds(b, 16)] = y * inv
            rz_v[pl.ds(b, 16)] = z * inv
            return carry

        lax.fori_loop(0, VSL // 16, norm_step, 0)
        pltpu.sync_copy(rx_v, svx.at[pl.ds(off, VSL)])
        pltpu.sync_copy(ry_v, svy.at[pl.ds(off, VSL)])
        pltpu.sync_copy(rz_v, svz.at[pl.ds(off, VSL)])
        plsc.subcore_barrier()
        pltpu.sync_copy(svx, ax_v)
        pltpu.sync_copy(svy, ay_v)
        pltpu.sync_copy(svz, az_v)

        def ds_step(k, carry):
            cnum, cden = carry
            b = k * 16
            dsi = ds_v[pl.ds(b, 16)]
            j = plsc.load_gather(amin_v, [dsi])
            nx = plsc.load_gather(ax_v, [j])
            ny = plsc.load_gather(ay_v, [j])
            nz = plsc.load_gather(az_v, [j])
            dx = plsc.load_gather(vx_v, [dsi]) - plsc.load_gather(vx_v, [j])
            dy = plsc.load_gather(vy_v, [dsi]) - plsc.load_gather(vy_v, [j])
            dz = plsc.load_gather(vz_v, [dsi]) - plsc.load_gather(vz_v, [j])
            ext = nx * dx + ny * dy + nz * dz >= 0.0
            ins = jnp.where(ext, 0, 1).astype(jnp.int32)
            buf_i[...] = ins
            pltpu.sync_copy(buf_i, sins.at[dsi])
            insf = ins.astype(jnp.float32)
            v2vd = plsc.load_gather(v2v_v, [dsi])
            gmd = plsc.load_gather(gmin_v, [dsi])
            val = 0.04 * (1.0 / (5.0 * gmd + 1.0)) * _tanh(v2vd / 0.04)
            cnum = cnum + val * (1.0 - insf)
            cden = cden + (1.0 - insf)
            return cnum, cden

        z16 = jnp.zeros((16,), jnp.float32)
        cnum, cden = lax.fori_loop(0, 4, ds_step, (z16, z16))
        plsc.subcore_barrier()
        pltpu.sync_copy(sins, inside_v)

        def slice_step(k, carry):
            inum, iden, anum, aden = carry
            b = w * VSL + k * 16
            pos = b + lax.iota(jnp.int32, 16)
            valid = pos < N
            insf = inside_v[pl.ds(b, 16)].astype(jnp.float32)
            v2vc = v2v_v[pl.ds(b, 16)]
            inum = inum + insf * _tanh(v2vc / 0.06)
            iden = iden + insf
            j = amin_v[pl.ds(b, 16)]
            dotn = (ax_v[pl.ds(b, 16)] * plsc.load_gather(ax_v, [j])
                    + ay_v[pl.ds(b, 16)] * plsc.load_gather(ay_v, [j])
                    + az_v[pl.ds(b, 16)] * plsc.load_gather(az_v, [j]))
            amask = (v2vc < 0.01) & valid
            anum = anum + jnp.where(amask, 1.0 + dotn, 0.0)
            aden = aden + jnp.where(amask, 1.0, 0.0)
            return inum, iden, anum, aden

        inum, iden, anum, aden = lax.fori_loop(
            0, VSL // 16, slice_step, (z16, z16, z16, z16))

        def hand_step(k, carry):
            lin, lid, lon, lod, rin, rid, ron, rod = carry
            pos = w * 64 + k * 16 + lax.iota(jnp.int32, 16)
            ok = jnp.where(pos < HA, 1.0, 0.0)
            posc = jnp.minimum(pos, HA - 1)
            li = plsc.load_gather(hp_v, [posc])
            ri = plsc.load_gather(hp_v, [posc + HA])
            lhv = plsc.load_gather(inside_v, [li]).astype(jnp.float32)
            rhv = plsc.load_gather(inside_v, [ri]).astype(jnp.float32)
            lv = plsc.load_gather(v2v_v, [li])
            rv = plsc.load_gather(v2v_v, [ri])
            lwc = 0.1 * (1.0 - plsc.load_gather(hw_v, [posc])) + 0.9
            rwc = 0.1 * (1.0 - plsc.load_gather(hw_v, [posc + HA])) + 0.9
            lin = lin + ok * lhv * _tanh(lv / 0.02)
            lid = lid + ok * lhv
            lon = lon + ok * (1.0 - lhv) * lwc * _tanh(lv / 0.01)
            lod = lod + ok * (1.0 - lhv)
            rin = rin + ok * rhv * _tanh(rv / 0.02)
            rid = rid + ok * rhv
            ron = ron + ok * (1.0 - rhv) * rwc * _tanh(rv / 0.01)
            rod = rod + ok * (1.0 - rhv)
            return lin, lid, lon, lod, rin, rid, ron, rod

        hand = lax.fori_loop(0, 4, hand_step, (z16,) * 8)

        lin, lid, lon, lod, rin, rid, ron, rod = hand
        nums = _lane_pack([jnp.sum(x) for x in
                           (inum, anum, cnum, lin, rin, lon, ron)])
        dens = _lane_pack([jnp.sum(x) for x in
                           (iden, aden, cden, lid, rid, lod, rod)])
        buf_f[...] = nums
        pltpu.sync_copy(buf_f, sparts.at[pl.ds(w * 32, 16)])
        buf_f[...] = dens
        pltpu.sync_copy(buf_f, sparts.at[pl.ds(w * 32 + 16, 16)])
        plsc.subcore_barrier()

        @pl.when(w == 0)
        def _final():
            pltpu.sync_copy(sparts, pbuf_v)
            numv = jnp.zeros((16,), jnp.float32)
            denv = jnp.zeros((16,), jnp.float32)
            for r in range(NT):
                numv = numv + pbuf_v[pl.ds(r * 32, 16)]
                denv = denv + pbuf_v[pl.ds(r * 32 + 16, 16)]
            quot = numv / jnp.maximum(denv, 1.0)
            coef = _lane_pack([0.5 * 0.07, 0.001, 5.0,
                               0.2 * 0.5 * 0.023, 0.2 * 0.5 * 0.023,
                               0.2 * 0.5 * 0.01, 0.2 * 0.5 * 0.01])
            loss = jnp.sum(coef * quot) + jnp.sum(base_v[...])
            lane = lax.iota(jnp.int32, 16)
            buf_f[...] = jnp.where(lane == 0, loss, 0.0)
            pltpu.sync_copy(buf_f, loss_o)
            pltpu.sync_copy(sins, inside_o)


def _sc_tail(vx, vy, vz, ff, v2v, amin, gmin, ds, hp, hw, base128, zf, zi):
    mesh = plsc.VectorSubcoreMesh(core_axis_name="c", subcore_axis_name="s")
    fn = pl.kernel(
        _sc_tail_body, mesh=mesh,
        compiler_params=pltpu.CompilerParams(needs_layout_passes=False),
        out_type=[jax.ShapeDtypeStruct((16,), jnp.float32),
                  jax.ShapeDtypeStruct((NP,), jnp.int32)],
        scratch_types=[
            pltpu.VMEM((NP,), jnp.float32),
            pltpu.VMEM((NP,), jnp.float32),
            pltpu.VMEM((NP,), jnp.float32),
            pltpu.VMEM((NP,), jnp.float32),
            pltpu.VMEM((NP,), jnp.float32),
            pltpu.VMEM((NP,), jnp.float32),
            pltpu.VMEM((NP,), jnp.float32),
            pltpu.VMEM((NP,), jnp.int32),
            pltpu.VMEM((NP,), jnp.float32),
            pltpu.VMEM((NP,), jnp.int32),
            pltpu.VMEM((3 * FSL,), jnp.int32),
            pltpu.VMEM((64,), jnp.int32),
            pltpu.VMEM((1600,), jnp.int32),
            pltpu.VMEM((1600,), jnp.float32),
            pltpu.VMEM((VSL,), jnp.float32),
            pltpu.VMEM((VSL,), jnp.float32),
            pltpu.VMEM((VSL,), jnp.float32),
            pltpu.VMEM((VSL,), jnp.float32),
            pltpu.VMEM((16,), jnp.int32),
            pltpu.VMEM((16,), jnp.float32),
            pltpu.VMEM((NT * 32,), jnp.float32),
            pltpu.VMEM((16,), jnp.float32),
            pltpu.VMEM_SHARED((NT * NP,), jnp.float32),
            pltpu.VMEM_SHARED((NT * NP,), jnp.float32),
            pltpu.VMEM_SHARED((NT * NP,), jnp.float32),
            pltpu.VMEM_SHARED((NP,), jnp.float32),
            pltpu.VMEM_SHARED((NP,), jnp.float32),
            pltpu.VMEM_SHARED((NP,), jnp.float32),
            pltpu.VMEM_SHARED((NP,), jnp.int32),
            pltpu.VMEM_SHARED((NT * 32,), jnp.float32),
        ],
    )
    return fn(vx, vy, vz, ff, v2v, amin, gmin, ds, hp, hw, base128, zf, zi)


def kernel(vertices, init_verts, body_pose, init_pose, left_hand_pose,
           right_hand_pose, geodist, hand_contact_prior_weights,
           ds, hand_contact_prior, faces, init_verts_in_contact):
    v = vertices[0]
    iv = init_verts[0]
    sel = jnp.zeros((1, N), jnp.float32).at[0, init_verts_in_contact].set(1.0)
    vmin, amin, gmin, vx, vy, vz, scal = _nn_pass(
        v, v.T, geodist, iv, sel, body_pose, init_pose,
        left_hand_pose, right_hand_pose)

    ff = jnp.zeros((3 * FP,), jnp.int32).at[:3 * F].set(faces.reshape(-1))
    hp = jnp.zeros((1600,), jnp.int32).at[:HCP].set(hand_contact_prior)
    hw = jnp.zeros((1600,), jnp.float32).at[:HCP].set(
        hand_contact_prior_weights)
    zf = jnp.zeros((NP,), jnp.float32)
    zi = jnp.zeros((NP,), jnp.int32)

    loss16, inside_np = _sc_tail(
        vx.reshape(NP), vy.reshape(NP), vz.reshape(NP), ff,
        vmin.reshape(NP), amin.reshape(NP), gmin.reshape(NP),
        ds.astype(jnp.int32), hp, hw, scal.reshape(128), zf, zi)
    return (loss16[0], inside_np[:N].astype(bool))


# speedup vs baseline: 1.1515x; 1.0721x over previous
import jax
import jax.numpy as jnp
from jax import lax
from jax.experimental import pallas as pl
from jax.experimental.pallas import tpu as pltpu
from jax.experimental.pallas import tpu_sc as plsc

N = 6890
F = 13776
NP = 6912
FP = 13824
NT = 16
VSL = NP // NT
FSL = FP // NT
HCP = 1556
HA = HCP // 2
BIG = 1e10
ROWS = 256


def _nn_body(v_ref, vt_ref, geo_ref, iv_ref, sel_ref, bp_ref, ip_ref,
             lh_ref, rh_ref, vmin_ref, amin_ref, gmin_ref,
             vx_ref, vy_ref, vz_ref, scal_ref):
    i = pl.program_id(0)
    vb = v_ref[...]
    vt = vt_ref[...]
    geo = geo_ref[...]
    sqi = (vb[:, 0:1] * vb[:, 0:1] + vb[:, 1:2] * vb[:, 1:2]
           + vb[:, 2:3] * vb[:, 2:3])
    sqj = (vt[0:1, :] * vt[0:1, :] + vt[1:2, :] * vt[1:2, :]
           + vt[2:3, :] * vt[2:3, :])
    dot = lax.dot_general(vb, vt, (((1,), (0,)), ((), ())),
                          preferred_element_type=jnp.float32)
    d2 = jnp.maximum((sqi + sqj) - 2.0 * dot, 0.0)
    masked = jnp.where(geo < 0.3, BIG, d2)
    m = jnp.min(masked, axis=1, keepdims=True)
    col = lax.broadcasted_iota(jnp.int32, masked.shape, 1)
    idx = jnp.min(jnp.where(masked == m, col, jnp.int32(2**30)),
                  axis=1, keepdims=True)
    row = i * ROWS + lax.broadcasted_iota(jnp.int32, (ROWS, 1), 0)
    ok = row < N
    vmin_ref[...] = jnp.where(
        ok, jnp.where(m >= 1e9, BIG, jnp.sqrt(m + 1e-12)), 0.0)
    amin_ref[...] = jnp.where(ok, idx, 0)
    gm = jnp.min(jnp.where(sel_ref[...] > 0.0, geo, BIG),
                 axis=1, keepdims=True)
    gmin_ref[...] = jnp.where(ok, gm, 0.0)
    vx_ref[...] = vb[:, 0:1]
    vy_ref[...] = vb[:, 1:2]
    vz_ref[...] = vb[:, 2:3]
    dv = iv_ref[...] - vb
    ov = jnp.sqrt(dv[:, 0:1] * dv[:, 0:1] + dv[:, 1:2] * dv[:, 1:2]
                  + dv[:, 2:3] * dv[:, 2:3])
    ow = (2.0 * gm) ** 2
    part = jnp.sum(jnp.where(ok, ov * ow, 0.0))
    lanev = lax.broadcasted_iota(jnp.int32, (1, 128), 1)

    @pl.when(i == 0)
    def _():
        bp = bp_ref[...]
        ip = ip_ref[...]
        pose = jnp.sum((bp - ip) ** 2)
        hand = 0.001 * (jnp.sum(lh_ref[...] ** 2) + jnp.sum(rh_ref[...] ** 2))
        scal_ref[...] = jnp.where(lanev == 0, pose + hand, 0.0)

    scal_ref[...] += jnp.where(lanev == 0, 2.0 * part, 0.0)


def _nn_pass(v, vt, geo, iv, sel, bp, ip, lh, rh):
    grid = NP // ROWS
    return pl.pallas_call(
        _nn_body,
        grid=(grid,),
        in_specs=[
            pl.BlockSpec((ROWS, 3), lambda i: (i, 0)),
            pl.BlockSpec((3, N), lambda i: (0, 0)),
            pl.BlockSpec((ROWS, N), lambda i: (i, 0)),
            pl.BlockSpec((ROWS, 3), lambda i: (i, 0)),
            pl.BlockSpec((1, N), lambda i: (0, 0)),
            pl.BlockSpec((1, 63), lambda i: (0, 0)),
            pl.BlockSpec((1, 63), lambda i: (0, 0)),
            pl.BlockSpec((1, 45), lambda i: (0, 0)),
            pl.BlockSpec((1, 45), lambda i: (0, 0)),
        ],
        out_specs=[
            pl.BlockSpec((ROWS, 1), lambda i: (i, 0)),
            pl.BlockSpec((ROWS, 1), lambda i: (i, 0)),
            pl.BlockSpec((ROWS, 1), lambda i: (i, 0)),
            pl.BlockSpec((ROWS, 1), lambda i: (i, 0)),
            pl.BlockSpec((ROWS, 1), lambda i: (i, 0)),
            pl.BlockSpec((ROWS, 1), lambda i: (i, 0)),
            pl.BlockSpec((1, 128), lambda i: (0, 0)),
        ],
        out_shape=[
            jax.ShapeDtypeStruct((NP, 1), jnp.float32),
            jax.ShapeDtypeStruct((NP, 1), jnp.int32),
            jax.ShapeDtypeStruct((NP, 1), jnp.float32),
            jax.ShapeDtypeStruct((NP, 1), jnp.float32),
            jax.ShapeDtypeStruct((NP, 1), jnp.float32),
            jax.ShapeDtypeStruct((NP, 1), jnp.float32),
            jax.ShapeDtypeStruct((1, 128), jnp.float32),
        ],
    )(v, vt, geo, iv, sel, bp, ip, lh, rh)


def _tanh(x):
    e = jnp.exp(2.0 * x)
    return 1.0 - 2.0 / (e + 1.0)


def _rsqrt(s):
    sc = jnp.maximum(s, 1e-30)
    i = plsc.bitcast(sc, jnp.int32)
    y = plsc.bitcast(jnp.int32(0x5F3759DF) - lax.shift_right_logical(i, 1),
                     jnp.float32)
    for _ in range(4):
        y = y * (1.5 - 0.5 * sc * y * y)
    return jnp.where(s < 1e-30, 0.0, y)


def _lane_pack(scalars):
    lane = lax.iota(jnp.int32, 16)
    out = jnp.zeros((16,), jnp.float32)
    for k, s in enumerate(scalars):
        out = out + jnp.where(lane == k, s, 0.0)
    return out


def _sc_tail_body(vx_h, vy_h, vz_h, ff_h, v2v_h, amin_h, gmin_h,
                  ds_h, hp_h, hw_h, base_h, zf_h, zi_h,
                  loss_o, inside_o,
                  vx_v, vy_v, vz_v, ax_v, ay_v, az_v,
                  v2v_v, amin_v, gmin_v, inside_v,
                  f_v, ds_v, hp_v, hw_v,
                  rx_v, ry_v, rz_v, tmp_v,
                  buf_i, buf_f, pbuf_v, base_v, sem,
                  sax, say, saz, svx, svy, svz, sins, sparts):
    cid = lax.axis_index("c")
    w = lax.axis_index("s")

    @pl.when(cid == 0)
    def _core0():
        hs = [
            pltpu.async_copy(vx_h, vx_v, sem),
            pltpu.async_copy(vy_h, vy_v, sem),
            pltpu.async_copy(vz_h, vz_v, sem),
            pltpu.async_copy(ff_h.at[pl.ds(w * (3 * FSL), 3 * FSL)], f_v, sem),
            pltpu.async_copy(v2v_h, v2v_v, sem),
            pltpu.async_copy(amin_h, amin_v, sem),
            pltpu.async_copy(gmin_h, gmin_v, sem),
            pltpu.async_copy(ds_h.at[pl.ds(w * 64, 64)], ds_v, sem),
            pltpu.async_copy(hp_h, hp_v, sem),
            pltpu.async_copy(hw_h, hw_v, sem),
            pltpu.async_copy(zf_h, ax_v, sem),
            pltpu.async_copy(zf_h, ay_v, sem),
            pltpu.async_copy(zf_h, az_v, sem),
        ]
        for h in hs:
            h.wait()

        @pl.when(w == 0)
        def _():
            pltpu.sync_copy(zi_h, sins)
            pltpu.sync_copy(base_h.at[pl.ds(0, 16)], base_v)

        i3 = lax.iota(jnp.int32, 16) * 3

        def face_step(k, carry):
            b3 = k * 48
            f0 = plsc.load_gather(f_v, [b3 + i3])
            f1 = plsc.load_gather(f_v, [b3 + i3 + 1])
            f2 = plsc.load_gather(f_v, [b3 + i3 + 2])
            x0 = plsc.load_gather(vx_v, [f0])
            y0 = plsc.load_gather(vy_v, [f0])
            z0 = plsc.load_gather(vz_v, [f0])
            x1 = plsc.load_gather(vx_v, [f1])
            y1 = plsc.load_gather(vy_v, [f1])
            z1 = plsc.load_gather(vz_v, [f1])
            x2 = plsc.load_gather(vx_v, [f2])
            y2 = plsc.load_gather(vy_v, [f2])
            z2 = plsc.load_gather(vz_v, [f2])
            e1x, e1y, e1z = x1 - x0, y1 - y0, z1 - z0
            e2x, e2y, e2z = x2 - x0, y2 - y0, z2 - z0
            cx = e1y * e2z - e1z * e2y
            cy = e1z * e2x - e1x * e2z
            cz = e1x * e2y - e1y * e2x
            plsc.addupdate_scatter(ax_v, [f0], cx)
            plsc.addupdate_scatter(ax_v, [f1], cx)
            plsc.addupdate_scatter(ax_v, [f2], cx)
            plsc.addupdate_scatter(ay_v, [f0], cy)
            plsc.addupdate_scatter(ay_v, [f1], cy)
            plsc.addupdate_scatter(ay_v, [f2], cy)
            plsc.addupdate_scatter(az_v, [f0], cz)
            plsc.addupdate_scatter(az_v, [f1], cz)
            plsc.addupdate_scatter(az_v, [f2], cz)
            return carry

        lax.fori_loop(0, FSL // 16, face_step, 0)
        pltpu.sync_copy(ax_v, sax.at[pl.ds(w * NP, NP)])
        pltpu.sync_copy(ay_v, say.at[pl.ds(w * NP, NP)])
        pltpu.sync_copy(az_v, saz.at[pl.ds(w * NP, NP)])
        plsc.subcore_barrier()

        off = w * VSL
        for comp, (src, dst) in enumerate(
                ((sax, rx_v), (say, ry_v), (saz, rz_v))):
            hs = [pltpu.async_copy(src.at[pl.ds(r * NP + off, VSL)],
                                   tmp_v.at[pl.ds(r * VSL, VSL)], sem)
                  for r in range(NT)]
            for h in hs:
                h.wait()

            def red_step(k, carry, dst=dst):
                b = k * 16
                acc = tmp_v[pl.ds(b, 16)]
                for r in range(1, NT):
                    acc = acc + tmp_v[pl.ds(r * VSL + b, 16)]
                dst[pl.ds(b, 16)] = acc
                return carry

            lax.fori_loop(0, VSL // 16, red_step, 0)

        def norm_step(k, carry):
            b = k * 16
            x = rx_v[pl.ds(b, 16)]
            y = ry_v[pl.ds(b, 16)]
            z = rz_v[pl.ds(b, 16)]
            s = x * x + y * y + z * z
            r = _rsqrt(s)
            inv = 1.0 / (s * r + 1e-12)
            rx_v[pl.ds(b, 16)] = x * inv
            ry_v[pl.ds(b, 16)] = y * inv
            rz_v[pl.ds(b, 16)] = z * inv
            return carry

        lax.fori_loop(0, VSL // 16, norm_step, 0)
        hs = [pltpu.async_copy(rx_v, svx.at[pl.ds(off, VSL)], sem),
              pltpu.async_copy(ry_v, svy.at[pl.ds(off, VSL)], sem),
              pltpu.async_copy(rz_v, svz.at[pl.ds(off, VSL)], sem)]
        for h in hs:
            h.wait()
        plsc.subcore_barrier()
        hs = [pltpu.async_copy(svx, ax_v, sem),
              pltpu.async_copy(svy, ay_v, sem),
              pltpu.async_copy(svz, az_v, sem)]
        for h in hs:
            h.wait()

        def ds_step(k, carry):
            cnum, cden = carry
            b = k * 16
            dsi = ds_v[pl.ds(b, 16)]
            j = plsc.load_gather(amin_v, [dsi])
            nx = plsc.load_gather(ax_v, [j])
            ny = plsc.load_gather(ay_v, [j])
            nz = plsc.load_gather(az_v, [j])
            dx = plsc.load_gather(vx_v, [dsi]) - plsc.load_gather(vx_v, [j])
            dy = plsc.load_gather(vy_v, [dsi]) - plsc.load_gather(vy_v, [j])
            dz = plsc.load_gather(vz_v, [dsi]) - plsc.load_gather(vz_v, [j])
            ext = nx * dx + ny * dy + nz * dz >= 0.0
            ins = jnp.where(ext, 0, 1).astype(jnp.int32)
            buf_i[...] = ins
            pltpu.sync_copy(buf_i, sins.at[dsi])
            insf = ins.astype(jnp.float32)
            v2vd = plsc.load_gather(v2v_v, [dsi])
            gmd = plsc.load_gather(gmin_v, [dsi])
            val = 0.04 * (1.0 / (5.0 * gmd + 1.0)) * _tanh(v2vd / 0.04)
            cnum = cnum + val * (1.0 - insf)
            cden = cden + (1.0 - insf)
            return cnum, cden

        z16 = jnp.zeros((16,), jnp.float32)
        cnum, cden = lax.fori_loop(0, 4, ds_step, (z16, z16))
        plsc.subcore_barrier()
        pltpu.sync_copy(sins, inside_v)

        def slice_step(k, carry):
            inum, iden, anum, aden = carry
            b = w * VSL + k * 16
            pos = b + lax.iota(jnp.int32, 16)
            valid = pos < N
            insf = inside_v[pl.ds(b, 16)].astype(jnp.float32)
            v2vc = v2v_v[pl.ds(b, 16)]
            inum = inum + insf * _tanh(v2vc / 0.06)
            iden = iden + insf
            j = amin_v[pl.ds(b, 16)]
            dotn = (ax_v[pl.ds(b, 16)] * plsc.load_gather(ax_v, [j])
                    + ay_v[pl.ds(b, 16)] * plsc.load_gather(ay_v, [j])
                    + az_v[pl.ds(b, 16)] * plsc.load_gather(az_v, [j]))
            amask = (v2vc < 0.01) & valid
            anum = anum + jnp.where(amask, 1.0 + dotn, 0.0)
            aden = aden + jnp.where(amask, 1.0, 0.0)
            return inum, iden, anum, aden

        inum, iden, anum, aden = lax.fori_loop(
            0, VSL // 16, slice_step, (z16, z16, z16, z16))

        def hand_step(k, carry):
            lin, lid, lon, lod, rin, rid, ron, rod = carry
            pos = w * 64 + k * 16 + lax.iota(jnp.int32, 16)
            ok = jnp.where(pos < HA, 1.0, 0.0)
            posc = jnp.minimum(pos, HA - 1)
            li = plsc.load_gather(hp_v, [posc])
            ri = plsc.load_gather(hp_v, [posc + HA])
            lhv = plsc.load_gather(inside_v, [li]).astype(jnp.float32)
            rhv = plsc.load_gather(inside_v, [ri]).astype(jnp.float32)
            lv = plsc.load_gather(v2v_v, [li])
            rv = plsc.load_gather(v2v_v, [ri])
            lwc = 0.1 * (1.0 - plsc.load_gather(hw_v, [posc])) + 0.9
            rwc = 0.1 * (1.0 - plsc.load_gather(hw_v, [posc + HA])) + 0.9
            lin = lin + ok * lhv * _tanh(lv / 0.02)
            lid = lid + ok * lhv
            lon = lon + ok * (1.0 - lhv) * lwc * _tanh(lv / 0.01)
            lod = lod + ok * (1.0 - lhv)
            rin = rin + ok * rhv * _tanh(rv / 0.02)
            rid = rid + ok * rhv
            ron = ron + ok * (1.0 - rhv) * rwc * _tanh(rv / 0.01)
            rod = rod + ok * (1.0 - rhv)
            return lin, lid, lon, lod, rin, rid, ron, rod

        hand = lax.fori_loop(0, 4, hand_step, (z16,) * 8)

        lin, lid, lon, lod, rin, rid, ron, rod = hand
        nums = _lane_pack([jnp.sum(x) for x in
                           (inum, anum, cnum, lin, rin, lon, ron)])
        dens = _lane_pack([jnp.sum(x) for x in
                           (iden, aden, cden, lid, rid, lod, rod)])
        buf_f[...] = nums
        pltpu.sync_copy(buf_f, sparts.at[pl.ds(w * 32, 16)])
        buf_f[...] = dens
        pltpu.sync_copy(buf_f, sparts.at[pl.ds(w * 32 + 16, 16)])
        plsc.subcore_barrier()

        @pl.when(w == 0)
        def _final():
            pltpu.sync_copy(sparts, pbuf_v)
            numv = jnp.zeros((16,), jnp.float32)
            denv = jnp.zeros((16,), jnp.float32)
            for r in range(NT):
                numv = numv + pbuf_v[pl.ds(r * 32, 16)]
                denv = denv + pbuf_v[pl.ds(r * 32 + 16, 16)]
            quot = numv / jnp.maximum(denv, 1.0)
            coef = _lane_pack([0.5 * 0.07, 0.001, 5.0,
                               0.2 * 0.5 * 0.023, 0.2 * 0.5 * 0.023,
                               0.2 * 0.5 * 0.01, 0.2 * 0.5 * 0.01])
            loss = jnp.sum(coef * quot) + jnp.sum(base_v[...])
            lane = lax.iota(jnp.int32, 16)
            buf_f[...] = jnp.where(lane == 0, loss, 0.0)
            pltpu.sync_copy(buf_f, loss_o)
            pltpu.sync_copy(sins, inside_o)


def _sc_tail(vx, vy, vz, ff, v2v, amin, gmin, ds, hp, hw, base128, zf, zi):
    mesh = plsc.VectorSubcoreMesh(core_axis_name="c", subcore_axis_name="s")
    fn = pl.kernel(
        _sc_tail_body, mesh=mesh,
        compiler_params=pltpu.CompilerParams(needs_layout_passes=False),
        out_type=[jax.ShapeDtypeStruct((16,), jnp.float32),
                  jax.ShapeDtypeStruct((NP,), jnp.int32)],
        scratch_types=[
            pltpu.VMEM((NP,), jnp.float32),
            pltpu.VMEM((NP,), jnp.float32),
            pltpu.VMEM((NP,), jnp.float32),
            pltpu.VMEM((NP,), jnp.float32),
            pltpu.VMEM((NP,), jnp.float32),
            pltpu.VMEM((NP,), jnp.float32),
            pltpu.VMEM((NP,), jnp.float32),
            pltpu.VMEM((NP,), jnp.int32),
            pltpu.VMEM((NP,), jnp.float32),
            pltpu.VMEM((NP,), jnp.int32),
            pltpu.VMEM((3 * FSL,), jnp.int32),
            pltpu.VMEM((64,), jnp.int32),
            pltpu.VMEM((1600,), jnp.int32),
            pltpu.VMEM((1600,), jnp.float32),
            pltpu.VMEM((VSL,), jnp.float32),
            pltpu.VMEM((VSL,), jnp.float32),
            pltpu.VMEM((VSL,), jnp.float32),
            pltpu.VMEM((NT * VSL,), jnp.float32),
            pltpu.VMEM((16,), jnp.int32),
            pltpu.VMEM((16,), jnp.float32),
            pltpu.VMEM((NT * 32,), jnp.float32),
            pltpu.VMEM((16,), jnp.float32),
            pltpu.SemaphoreType.DMA,
            pltpu.VMEM_SHARED((NT * NP,), jnp.float32),
            pltpu.VMEM_SHARED((NT * NP,), jnp.float32),
            pltpu.VMEM_SHARED((NT * NP,), jnp.float32),
            pltpu.VMEM_SHARED((NP,), jnp.float32),
            pltpu.VMEM_SHARED((NP,), jnp.float32),
            pltpu.VMEM_SHARED((NP,), jnp.float32),
            pltpu.VMEM_SHARED((NP,), jnp.int32),
            pltpu.VMEM_SHARED((NT * 32,), jnp.float32),
        ],
    )
    return fn(vx, vy, vz, ff, v2v, amin, gmin, ds, hp, hw, base128, zf, zi)


def kernel(vertices, init_verts, body_pose, init_pose, left_hand_pose,
           right_hand_pose, geodist, hand_contact_prior_weights,
           ds, hand_contact_prior, faces, init_verts_in_contact):
    v = vertices[0]
    iv = init_verts[0]
    sel = jnp.zeros((1, N), jnp.float32).at[0, init_verts_in_contact].set(1.0)
    vmin, amin, gmin, vx, vy, vz, scal = _nn_pass(
        v, v.T, geodist, iv, sel, body_pose, init_pose,
        left_hand_pose, right_hand_pose)

    ff = jnp.zeros((3 * FP,), jnp.int32).at[:3 * F].set(faces.reshape(-1))
    hp = jnp.zeros((1600,), jnp.int32).at[:HCP].set(hand_contact_prior)
    hw = jnp.zeros((1600,), jnp.float32).at[:HCP].set(
        hand_contact_prior_weights)
    zf = jnp.zeros((NP,), jnp.float32)
    zi = jnp.zeros((NP,), jnp.int32)

    loss16, inside_np = _sc_tail(
        vx.reshape(NP), vy.reshape(NP), vz.reshape(NP), ff,
        vmin.reshape(NP), amin.reshape(NP), gmin.reshape(NP),
        ds.astype(jnp.int32), hp, hw, scal.reshape(128), zf, zi)
    return (loss16[0], inside_np[:N].astype(bool))
